# baseline (speedup 1.0000x reference)
import functools
import numpy as np
import jax
import jax.numpy as jnp
from jax import lax
from jax.experimental import pallas as pl
from jax.experimental.pallas import tpu as pltpu

C = 96
PS = 16
K = 12
NC = 8
NI = 3
P = 576
PROW = 24
NBLK = 24

_HI = jax.lax.Precision.HIGHEST


def _lrelu(x):
    return jnp.where(x > 0, x, 0.2 * x)


def _erf(z):
    s = jnp.sign(z)
    a = jnp.abs(z)
    t = 1.0 / (1.0 + 0.3275911 * a)
    poly = t * (0.254829592 + t * (-0.284496736 + t * (1.421413741
               + t * (-1.453152027 + t * 1.061405429))))
    return s * (1.0 - poly * jnp.exp(-a * a))


def _gelu(x):
    return 0.5 * x * (1.0 + _erf(x * 0.7071067811865476))


def _ln_rows(x, w, b):
    mu = jnp.mean(x, axis=-1, keepdims=True)
    var = jnp.mean((x - mu) ** 2, axis=-1, keepdims=True)
    return (x - mu) / jnp.sqrt(var + 1e-5) * w + b


def _dot_t(x, w, prec=None):
    return lax.dot_general(x, w, (((1,), (1,)), ((), ())),
                           precision=prec, preferred_element_type=jnp.float32)


def _dot(x, w, prec=None):
    return lax.dot_general(x, w, (((1,), (0,)), ((), ())),
                           precision=prec, preferred_element_type=jnp.float32)


def _pool_phase(i, fa_ref, nodes_scr):
    x = fa_ref[0]
    xm = jnp.sum(x, axis=1)
    r = lax.broadcasted_iota(jnp.int32, (PROW, 384), 0)
    c = lax.broadcasted_iota(jnp.int32, (PROW, 384), 1)
    q = jnp.where(c // PS == r, 1.0 / (PS * PS), 0.0)
    nodes_scr[pl.ds(i, 1)] = _dot_t(q, xm, _HI)[None]


def _graph_phase(nodes_scr, we1_ref, be1_ref, we2_ref, be2_ref, lng_w_ref,
                 lng_b_ref, wpx_ref, bpx_ref, wnp_ref, bnp_ref, lam_ref,
                 lna_w_ref, lna_b_ref, wpr_ref, bpr_ref, wrf_ref, brf_ref,
                 wfc_ref, bfc_ref, g_scr):
    x = nodes_scr[...].reshape(P, C)
    we1 = we1_ref[...]
    wc = we1[:, :C]
    wn = we1[:, C:]

    nrm = jnp.sqrt(jnp.sum(x * x, axis=1, keepdims=True))
    nn = x / jnp.maximum(nrm, 1e-12)
    sim = _dot_t(nn, nn, _HI)
    rr = lax.broadcasted_iota(jnp.int32, (P, P), 0)
    cc = lax.broadcasted_iota(jnp.int32, (P, P), 1)
    minv = jnp.int32(-2147483647 - 1)
    e = (sim * 1048576.0).astype(jnp.int32) * 1024 + (1023 - cc)
    e = jnp.where(rr == cc, minv, e)

    a_proj = _dot_t(x, wc - wn) + be1_ref[...]
    b_proj = _dot_t(x, wn)
    b16 = b_proj.astype(jnp.bfloat16)
    we2_16 = we2_ref[...].astype(jnp.bfloat16)
    be2 = be2_ref[...]

    acc = jnp.zeros((P, C), jnp.float32)
    for _ in range(K):
        emax = jnp.max(e, axis=1, keepdims=True)
        sel = e == emax
        e = jnp.where(sel, minv, e)
        oh = sel.astype(jnp.bfloat16)
        bq = _dot(oh, b16)
        h1 = _lrelu(a_proj + bq).astype(jnp.bfloat16)
        acc = acc + _lrelu(_dot_t(h1, we2_16) + be2)

    fg = _ln_rows(acc * (1.0 / K), lng_w_ref[...], lng_b_ref[...])

    ones_row = jnp.ones((1, C), jnp.float32)
    wpx_outer = lax.dot_general(wpx_ref[...], ones_row,
                                (((0,), (0,)), ((), ())),
                                preferred_element_type=jnp.float32)
    sp = _gelu(_dot(fg, wpx_outer) + bpx_ref[...])
    sn = jnp.mean(fg, axis=0, keepdims=True)
    sn = _gelu(_dot_t(sn, wnp_ref[...]) + bnp_ref[...])
    fp = _ln_rows(lam_ref[...] * (sp + sn), lna_w_ref[...], lna_b_ref[...])

    n2 = _dot_t(fp, wpr_ref[...]) + bpr_ref[...]
    n2n = jnp.sqrt(jnp.sum(n2 * n2, axis=1, keepdims=True))
    nn2 = n2 / jnp.maximum(n2n, 1e-12)
    cidx = [0, 82, 164, 246, 328, 410, 492, 575]
    centers = jnp.concatenate([n2[i:i + 1] for i in cidx], axis=0)
    c8 = lax.broadcasted_iota(jnp.int32, (P, NC), 1)
    ones_p = jnp.ones((P, 1), jnp.float32)
    for _ in range(NI):
        cn = jnp.sqrt(jnp.sum(centers * centers, axis=1, keepdims=True))
        cnn = centers / jnp.maximum(cn, 1e-12)
        s2 = _dot_t(nn2, cnn, _HI)
        m = jnp.max(s2, axis=1, keepdims=True)
        amin = jnp.min(jnp.where(s2 == m, c8, NC), axis=1, keepdims=True)
        oh = (c8 == amin).astype(jnp.float32)
        cnt = lax.dot_general(oh, ones_p, (((0,), (0,)), ((), ())),
                              preferred_element_type=jnp.float32)
        csum = lax.dot_general(oh, n2, (((0,), (0,)), ((), ())),
                               preferred_element_type=jnp.float32)
        centers = csum / jnp.maximum(cnt, 1.0)
    cn = jnp.sqrt(jnp.sum(centers * centers, axis=1, keepdims=True))
    cnn = centers / jnp.maximum(cn, 1e-12)
    s2 = _dot_t(nn2, cnn, _HI) * 10.0
    s2 = s2 - jnp.max(s2, axis=1, keepdims=True)
    ex = jnp.exp(s2)
    wts = ex / jnp.sum(ex, axis=1, keepdims=True)
    cl = _dot(wts, centers)

    fo = _dot_t(cl + fp, wrf_ref[...]) + brf_ref[...]
    wfc = wfc_ref[...]
    bfc_col = bfc_ref[...]
    for i in range(NBLK):
        blk = lax.dot_general(wfc, fo[i * PROW:(i + 1) * PROW, :],
                              (((1,), (1,)), ((), ())),
                              preferred_element_type=jnp.float32)
        g_scr[i] = blk + bfc_col


def _conv_phase(i, fa_ref, wfc_ref, g_scr, out_ref):
    x = fa_ref[0].astype(jnp.bfloat16)
    gt = g_scr[pl.ds(i, 1)][0]
    w = wfc_ref[...].astype(jnp.bfloat16)
    c = lax.broadcasted_iota(jnp.int32, (PROW, 384), 1)
    r = lax.broadcasted_iota(jnp.int32, (PROW, 384), 0)
    rep = jnp.where(c // PS == r, 1.0, 0.0)
    up = _dot(gt, rep)
    for hh in range(PS):
        out_ref[0, :, hh, :] = _dot(w, x[:, hh, :]) + up


def _fused_body(fa_ref, we1_ref, be1_ref, we2_ref, be2_ref, lng_w_ref,
                lng_b_ref, wpx_ref, bpx_ref, wnp_ref, bnp_ref, lam_ref,
                lna_w_ref, lna_b_ref, wpr_ref, bpr_ref, wrf_ref, brf_ref,
                wfc_ref, bfc_ref, out_ref, nodes_scr, g_scr):
    i = pl.program_id(1)

    @pl.when(i < NBLK)
    def _():
        _pool_phase(i, fa_ref, nodes_scr)

    @pl.when(i == NBLK)
    def _():
        _graph_phase(nodes_scr, we1_ref, be1_ref, we2_ref, be2_ref, lng_w_ref,
                     lng_b_ref, wpx_ref, bpx_ref, wnp_ref, bnp_ref, lam_ref,
                     lna_w_ref, lna_b_ref, wpr_ref, bpr_ref, wrf_ref, brf_ref,
                     wfc_ref, bfc_ref, g_scr)

    @pl.when(i > NBLK)
    def _():
        _conv_phase(i - NBLK - 1, fa_ref, wfc_ref, g_scr, out_ref)


def kernel(F_a, W_e1, b_e1, W_e2, b_e2, ln_g_w, ln_g_b, W_px, b_px, W_np,
           b_np, lambda_n, ln_a_w, ln_a_b, W_pr, b_pr, W_rf, b_rf, W_fc,
           b_fc):
    B, Cc, H, W = F_a.shape
    nblk = H // PS

    wfull = lambda s: pl.BlockSpec(s, lambda *_: tuple(0 for _ in s))
    row = lambda v: v.reshape(1, -1)

    def fa_idx(b, i):
        j = jnp.where(i < nblk, i, jnp.maximum(i - nblk - 1, 0))
        return (b, 0, j, 0)

    def out_idx(b, i):
        return (b, 0, jnp.maximum(i - nblk - 1, 0), 0)

    out = pl.pallas_call(
        _fused_body,
        grid=(B, 2 * nblk + 1),
        in_specs=[pl.BlockSpec((1, Cc, PS, W), fa_idx)]
        + [wfull(s) for s in [(Cc, 2 * Cc), (1, Cc), (Cc, Cc), (1, Cc),
                              (1, Cc), (1, Cc), (1, Cc), (1, Cc), (Cc, Cc),
                              (1, Cc), (1, Cc), (1, Cc), (1, Cc), (Cc, Cc),
                              (1, Cc), (Cc, Cc), (1, Cc), (Cc, Cc),
                              (Cc, 1)]],
        out_specs=pl.BlockSpec((1, Cc, PS, W), out_idx),
        out_shape=jax.ShapeDtypeStruct((B, Cc, H, W), jnp.float32),
        scratch_shapes=[
            pltpu.VMEM((nblk, PROW, Cc), jnp.float32),
            pltpu.VMEM((nblk, Cc, PROW), jnp.float32),
        ],
    )(F_a, W_e1, row(b_e1), W_e2, row(b_e2), row(ln_g_w), row(ln_g_b),
      W_px, jnp.broadcast_to(b_px.reshape(1, 1), (1, Cc)), W_np, row(b_np),
      lambda_n.reshape(1, Cc), row(ln_a_w), row(ln_a_b), W_pr, row(b_pr),
      W_rf, row(b_rf), W_fc.reshape(Cc, Cc), b_fc.reshape(Cc, 1))

    return out


# speedup vs baseline: 1.0148x; 1.0148x over previous
import functools
import numpy as np
import jax
import jax.numpy as jnp
from jax import lax
from jax.experimental import pallas as pl
from jax.experimental.pallas import tpu as pltpu

C = 96
PS = 16
K = 12
NC = 8
NI = 3
P = 576
PROW = 24
NBLK = 24

_HI = jax.lax.Precision.HIGHEST


def _lrelu(x):
    return jnp.where(x > 0, x, 0.2 * x)


def _erf(z):
    s = jnp.sign(z)
    a = jnp.abs(z)
    t = 1.0 / (1.0 + 0.3275911 * a)
    poly = t * (0.254829592 + t * (-0.284496736 + t * (1.421413741
               + t * (-1.453152027 + t * 1.061405429))))
    return s * (1.0 - poly * jnp.exp(-a * a))


def _gelu(x):
    return 0.5 * x * (1.0 + _erf(x * 0.7071067811865476))


def _ln_rows(x, w, b):
    mu = jnp.mean(x, axis=-1, keepdims=True)
    var = jnp.mean((x - mu) ** 2, axis=-1, keepdims=True)
    return (x - mu) / jnp.sqrt(var + 1e-5) * w + b


def _dot_t(x, w, prec=None):
    return lax.dot_general(x, w, (((1,), (1,)), ((), ())),
                           precision=prec, preferred_element_type=jnp.float32)


def _dot(x, w, prec=None):
    return lax.dot_general(x, w, (((1,), (0,)), ((), ())),
                           precision=prec, preferred_element_type=jnp.float32)


def _pool_phase(i, fa_ref, nodes_scr):
    x = fa_ref[0]
    xm = jnp.sum(x, axis=1)
    r = lax.broadcasted_iota(jnp.int32, (PROW, 384), 0)
    c = lax.broadcasted_iota(jnp.int32, (PROW, 384), 1)
    q = jnp.where(c // PS == r, 1.0 / (PS * PS), 0.0)
    nodes_scr[pl.ds(i, 1)] = _dot_t(q, xm, _HI)[None]


def _graph_phase(nodes_scr, we1_ref, be1_ref, we2_ref, be2_ref, lng_w_ref,
                 lng_b_ref, wpx_ref, bpx_ref, wnp_ref, bnp_ref, lam_ref,
                 lna_w_ref, lna_b_ref, wpr_ref, bpr_ref, wrf_ref, brf_ref,
                 wfc_ref, bfc_ref, g_scr):
    x = nodes_scr[...].reshape(P, C)
    we1 = we1_ref[...]
    wc = we1[:, :C]
    wn = we1[:, C:]

    nrm = jnp.sqrt(jnp.sum(x * x, axis=1, keepdims=True))
    nn = x / jnp.maximum(nrm, 1e-12)
    sim = _dot_t(nn, nn, _HI)
    rr = lax.broadcasted_iota(jnp.int32, (P, P), 0)
    cc = lax.broadcasted_iota(jnp.int32, (P, P), 1)
    minv = jnp.int32(-2147483647 - 1)
    e = (sim * 1048576.0).astype(jnp.int32) * 1024 + (1023 - cc)
    e = jnp.where(rr == cc, minv, e)

    a_proj = _dot_t(x, wc - wn) + be1_ref[...]
    b_proj = _dot_t(x, wn)
    b16 = b_proj.astype(jnp.bfloat16)
    we2_16 = we2_ref[...].astype(jnp.bfloat16)
    be2 = be2_ref[...]

    acc = jnp.zeros((P, C), jnp.float32)
    for _ in range(K):
        emax = jnp.max(e, axis=1, keepdims=True)
        sel = e == emax
        e = jnp.where(sel, minv, e)
        oh = sel.astype(jnp.bfloat16)
        bq = _dot(oh, b16)
        h1 = _lrelu(a_proj + bq).astype(jnp.bfloat16)
        acc = acc + _lrelu(_dot_t(h1, we2_16) + be2)

    fg = _ln_rows(acc * (1.0 / K), lng_w_ref[...], lng_b_ref[...])

    ones_row = jnp.ones((1, C), jnp.float32)
    wpx_outer = lax.dot_general(wpx_ref[...], ones_row,
                                (((0,), (0,)), ((), ())),
                                preferred_element_type=jnp.float32)
    sp = _gelu(_dot(fg, wpx_outer) + bpx_ref[...])
    sn = jnp.mean(fg, axis=0, keepdims=True)
    sn = _gelu(_dot_t(sn, wnp_ref[...]) + bnp_ref[...])
    fp = _ln_rows(lam_ref[...] * (sp + sn), lna_w_ref[...], lna_b_ref[...])

    n2 = _dot_t(fp, wpr_ref[...]) + bpr_ref[...]
    n2n = jnp.sqrt(jnp.sum(n2 * n2, axis=1, keepdims=True))
    nn2 = n2 / jnp.maximum(n2n, 1e-12)
    cidx = [0, 82, 164, 246, 328, 410, 492, 575]
    centers = jnp.concatenate([n2[i:i + 1] for i in cidx], axis=0)
    c8 = lax.broadcasted_iota(jnp.int32, (P, NC), 1)
    ones_p = jnp.ones((P, 1), jnp.float32)
    for _ in range(NI):
        cn = jnp.sqrt(jnp.sum(centers * centers, axis=1, keepdims=True))
        cnn = centers / jnp.maximum(cn, 1e-12)
        s2 = _dot_t(nn2, cnn, _HI)
        m = jnp.max(s2, axis=1, keepdims=True)
        amin = jnp.min(jnp.where(s2 == m, c8, NC), axis=1, keepdims=True)
        oh = (c8 == amin).astype(jnp.float32)
        cnt = lax.dot_general(oh, ones_p, (((0,), (0,)), ((), ())),
                              preferred_element_type=jnp.float32)
        csum = lax.dot_general(oh, n2, (((0,), (0,)), ((), ())),
                               preferred_element_type=jnp.float32)
        centers = csum / jnp.maximum(cnt, 1.0)
    cn = jnp.sqrt(jnp.sum(centers * centers, axis=1, keepdims=True))
    cnn = centers / jnp.maximum(cn, 1e-12)
    s2 = _dot_t(nn2, cnn, _HI) * 10.0
    s2 = s2 - jnp.max(s2, axis=1, keepdims=True)
    ex = jnp.exp(s2)
    wts = ex / jnp.sum(ex, axis=1, keepdims=True)
    cl = _dot(wts, centers)

    fo = _dot_t(cl + fp, wrf_ref[...]) + brf_ref[...]
    wfc = wfc_ref[...]
    bfc_col = bfc_ref[...]
    for i in range(NBLK):
        blk = lax.dot_general(wfc, fo[i * PROW:(i + 1) * PROW, :],
                              (((1,), (1,)), ((), ())),
                              preferred_element_type=jnp.float32)
        g_scr[i] = blk + bfc_col


def _conv_phase(i, fa_ref, wfc_ref, g_scr, out_ref):
    x = fa_ref[0]
    gt = g_scr[pl.ds(i, 1)][0]
    w = wfc_ref[...]
    c = lax.broadcasted_iota(jnp.int32, (PROW, 384), 1)
    r = lax.broadcasted_iota(jnp.int32, (PROW, 384), 0)
    rep = jnp.where(c // PS == r, 1.0, 0.0)
    up = _dot(gt, rep)
    for hh in range(PS):
        out_ref[0, :, hh, :] = _dot(w, x[:, hh, :]) + up


def _fused_body(fa_ref, we1_ref, be1_ref, we2_ref, be2_ref, lng_w_ref,
                lng_b_ref, wpx_ref, bpx_ref, wnp_ref, bnp_ref, lam_ref,
                lna_w_ref, lna_b_ref, wpr_ref, bpr_ref, wrf_ref, brf_ref,
                wfc_ref, bfc_ref, out_ref, nodes_scr, g_scr):
    i = pl.program_id(1)

    @pl.when(i < NBLK)
    def _():
        _pool_phase(i, fa_ref, nodes_scr)

    @pl.when(i == NBLK)
    def _():
        _graph_phase(nodes_scr, we1_ref, be1_ref, we2_ref, be2_ref, lng_w_ref,
                     lng_b_ref, wpx_ref, bpx_ref, wnp_ref, bnp_ref, lam_ref,
                     lna_w_ref, lna_b_ref, wpr_ref, bpr_ref, wrf_ref, brf_ref,
                     wfc_ref, bfc_ref, g_scr)

    @pl.when(i > NBLK)
    def _():
        _conv_phase(i - NBLK - 1, fa_ref, wfc_ref, g_scr, out_ref)


def kernel(F_a, W_e1, b_e1, W_e2, b_e2, ln_g_w, ln_g_b, W_px, b_px, W_np,
           b_np, lambda_n, ln_a_w, ln_a_b, W_pr, b_pr, W_rf, b_rf, W_fc,
           b_fc):
    B, Cc, H, W = F_a.shape
    nblk = H // PS

    wfull = lambda s: pl.BlockSpec(s, lambda *_: tuple(0 for _ in s))
    row = lambda v: v.reshape(1, -1)

    def fa_idx(b, i):
        j = jnp.where(i < nblk, i, jnp.maximum(i - nblk - 1, 0))
        return (b, 0, j, 0)

    def out_idx(b, i):
        return (b, 0, jnp.maximum(i - nblk - 1, 0), 0)

    out = pl.pallas_call(
        _fused_body,
        grid=(B, 2 * nblk + 1),
        in_specs=[pl.BlockSpec((1, Cc, PS, W), fa_idx)]
        + [wfull(s) for s in [(Cc, 2 * Cc), (1, Cc), (Cc, Cc), (1, Cc),
                              (1, Cc), (1, Cc), (1, Cc), (1, Cc), (Cc, Cc),
                              (1, Cc), (1, Cc), (1, Cc), (1, Cc), (Cc, Cc),
                              (1, Cc), (Cc, Cc), (1, Cc), (Cc, Cc),
                              (Cc, 1)]],
        out_specs=pl.BlockSpec((1, Cc, PS, W), out_idx),
        out_shape=jax.ShapeDtypeStruct((B, Cc, H, W), jnp.float32),
        scratch_shapes=[
            pltpu.VMEM((nblk, PROW, Cc), jnp.float32),
            pltpu.VMEM((nblk, Cc, PROW), jnp.float32),
        ],
    )(F_a, W_e1, row(b_e1), W_e2, row(b_e2), row(ln_g_w), row(ln_g_b),
      W_px, jnp.broadcast_to(b_px.reshape(1, 1), (1, Cc)), W_np, row(b_np),
      lambda_n.reshape(1, Cc), row(ln_a_w), row(ln_a_b), W_pr, row(b_pr),
      W_rf, row(b_rf), W_fc.reshape(Cc, Cc), b_fc.reshape(Cc, 1))

    return out


# speedup vs baseline: 1.0506x; 1.0353x over previous
import functools
import numpy as np
import jax
import jax.numpy as jnp
from jax import lax
from jax.experimental import pallas as pl
from jax.experimental.pallas import tpu as pltpu

C = 96
PS = 16
K = 12
NC = 8
NI = 3
P = 576
PROW = 24
NBLK = 24
NBAND = 12

_HI = jax.lax.Precision.HIGHEST


def _lrelu(x):
    return jnp.where(x > 0, x, 0.2 * x)


def _erf(z):
    s = jnp.sign(z)
    a = jnp.abs(z)
    t = 1.0 / (1.0 + 0.3275911 * a)
    poly = t * (0.254829592 + t * (-0.284496736 + t * (1.421413741
               + t * (-1.453152027 + t * 1.061405429))))
    return s * (1.0 - poly * jnp.exp(-a * a))


def _gelu(x):
    return 0.5 * x * (1.0 + _erf(x * 0.7071067811865476))


def _ln_rows(x, w, b):
    mu = jnp.mean(x, axis=-1, keepdims=True)
    var = jnp.mean((x - mu) ** 2, axis=-1, keepdims=True)
    return (x - mu) / jnp.sqrt(var + 1e-5) * w + b


def _dot_t(x, w, prec=None):
    return lax.dot_general(x, w, (((1,), (1,)), ((), ())),
                           precision=prec, preferred_element_type=jnp.float32)


def _dot(x, w, prec=None):
    return lax.dot_general(x, w, (((1,), (0,)), ((), ())),
                           precision=prec, preferred_element_type=jnp.float32)


def _pool_phase(i, fa_ref, nodes_scr):
    x = fa_ref[0]
    r = lax.broadcasted_iota(jnp.int32, (PROW, 384), 0)
    c = lax.broadcasted_iota(jnp.int32, (PROW, 384), 1)
    q = jnp.where(c // PS == r, 1.0 / (PS * PS), 0.0)
    for h in range(2):
        xm = jnp.sum(x[:, h * PS:(h + 1) * PS, :], axis=1)
        nodes_scr[pl.ds(2 * i + h, 1)] = _dot_t(q, xm, _HI)[None]


def _graph_phase(nodes_scr, we1_ref, be1_ref, we2_ref, be2_ref, lng_w_ref,
                 lng_b_ref, wpx_ref, bpx_ref, wnp_ref, bnp_ref, lam_ref,
                 lna_w_ref, lna_b_ref, wpr_ref, bpr_ref, wrf_ref, brf_ref,
                 wfc_ref, bfc_ref, g_scr):
    x = nodes_scr[...].reshape(P, C)
    we1 = we1_ref[...]
    wc = we1[:, :C]
    wn = we1[:, C:]

    nrm = jnp.sqrt(jnp.sum(x * x, axis=1, keepdims=True))
    nn = x / jnp.maximum(nrm, 1e-12)
    sim = _dot_t(nn, nn, _HI)
    rr = lax.broadcasted_iota(jnp.int32, (P, P), 0)
    cc = lax.broadcasted_iota(jnp.int32, (P, P), 1)
    minv = jnp.int32(-2147483647 - 1)
    e = (sim * 1048576.0).astype(jnp.int32) * 1024 + (1023 - cc)
    e = jnp.where(rr == cc, minv, e)

    a_proj = _dot_t(x, wc - wn) + be1_ref[...]
    b_proj = _dot_t(x, wn)
    b16 = b_proj.astype(jnp.bfloat16)
    we2_16 = we2_ref[...].astype(jnp.bfloat16)
    be2 = be2_ref[...]

    acc = jnp.zeros((P, C), jnp.float32)
    for _ in range(K):
        emax = jnp.max(e, axis=1, keepdims=True)
        sel = e == emax
        e = jnp.where(sel, minv, e)
        oh = sel.astype(jnp.bfloat16)
        bq = _dot(oh, b16)
        h1 = _lrelu(a_proj + bq).astype(jnp.bfloat16)
        acc = acc + _lrelu(_dot_t(h1, we2_16) + be2)

    fg = _ln_rows(acc * (1.0 / K), lng_w_ref[...], lng_b_ref[...])

    ones_row = jnp.ones((1, C), jnp.float32)
    wpx_outer = lax.dot_general(wpx_ref[...], ones_row,
                                (((0,), (0,)), ((), ())),
                                preferred_element_type=jnp.float32)
    sp = _gelu(_dot(fg, wpx_outer) + bpx_ref[...])
    sn = jnp.mean(fg, axis=0, keepdims=True)
    sn = _gelu(_dot_t(sn, wnp_ref[...]) + bnp_ref[...])
    fp = _ln_rows(lam_ref[...] * (sp + sn), lna_w_ref[...], lna_b_ref[...])

    n2 = _dot_t(fp, wpr_ref[...]) + bpr_ref[...]
    n2n = jnp.sqrt(jnp.sum(n2 * n2, axis=1, keepdims=True))
    nn2 = n2 / jnp.maximum(n2n, 1e-12)
    cidx = [0, 82, 164, 246, 328, 410, 492, 575]
    centers = jnp.concatenate([n2[i:i + 1] for i in cidx], axis=0)
    c8 = lax.broadcasted_iota(jnp.int32, (P, NC), 1)
    ones_p = jnp.ones((P, 1), jnp.float32)
    for _ in range(NI):
        cn = jnp.sqrt(jnp.sum(centers * centers, axis=1, keepdims=True))
        cnn = centers / jnp.maximum(cn, 1e-12)
        s2 = _dot_t(nn2, cnn, _HI)
        m = jnp.max(s2, axis=1, keepdims=True)
        amin = jnp.min(jnp.where(s2 == m, c8, NC), axis=1, keepdims=True)
        oh = (c8 == amin).astype(jnp.float32)
        cnt = lax.dot_general(oh, ones_p, (((0,), (0,)), ((), ())),
                              preferred_element_type=jnp.float32)
        csum = lax.dot_general(oh, n2, (((0,), (0,)), ((), ())),
                               preferred_element_type=jnp.float32)
        centers = csum / jnp.maximum(cnt, 1.0)
    cn = jnp.sqrt(jnp.sum(centers * centers, axis=1, keepdims=True))
    cnn = centers / jnp.maximum(cn, 1e-12)
    s2 = _dot_t(nn2, cnn, _HI) * 10.0
    s2 = s2 - jnp.max(s2, axis=1, keepdims=True)
    ex = jnp.exp(s2)
    wts = ex / jnp.sum(ex, axis=1, keepdims=True)
    cl = _dot(wts, centers)

    fo = _dot_t(cl + fp, wrf_ref[...]) + brf_ref[...]
    wfc = wfc_ref[...]
    bfc_col = bfc_ref[...]
    for i in range(NBLK):
        blk = lax.dot_general(wfc, fo[i * PROW:(i + 1) * PROW, :],
                              (((1,), (1,)), ((), ())),
                              preferred_element_type=jnp.float32)
        g_scr[i] = blk + bfc_col


def _conv_phase(i, fa_ref, wfc_ref, g_scr, out_ref):
    x = fa_ref[0]
    w = wfc_ref[...]
    c = lax.broadcasted_iota(jnp.int32, (PROW, 384), 1)
    r = lax.broadcasted_iota(jnp.int32, (PROW, 384), 0)
    rep = jnp.where(c // PS == r, 1.0, 0.0)
    for h in range(2):
        gt = g_scr[pl.ds(2 * i + h, 1)][0]
        up = _dot(gt, rep)
        for hh in range(PS):
            row = h * PS + hh
            out_ref[0, :, row, :] = _dot(w, x[:, row, :]) + up


def _fused_body(fa_ref, we1_ref, be1_ref, we2_ref, be2_ref, lng_w_ref,
                lng_b_ref, wpx_ref, bpx_ref, wnp_ref, bnp_ref, lam_ref,
                lna_w_ref, lna_b_ref, wpr_ref, bpr_ref, wrf_ref, brf_ref,
                wfc_ref, bfc_ref, out_ref, nodes_scr, g_scr):
    i = pl.program_id(1)

    @pl.when(i < NBAND)
    def _():
        _pool_phase(i, fa_ref, nodes_scr)

    @pl.when(i == NBAND)
    def _():
        _graph_phase(nodes_scr, we1_ref, be1_ref, we2_ref, be2_ref, lng_w_ref,
                     lng_b_ref, wpx_ref, bpx_ref, wnp_ref, bnp_ref, lam_ref,
                     lna_w_ref, lna_b_ref, wpr_ref, bpr_ref, wrf_ref, brf_ref,
                     wfc_ref, bfc_ref, g_scr)

    @pl.when(i > NBAND)
    def _():
        _conv_phase(i - NBAND - 1, fa_ref, wfc_ref, g_scr, out_ref)


def kernel(F_a, W_e1, b_e1, W_e2, b_e2, ln_g_w, ln_g_b, W_px, b_px, W_np,
           b_np, lambda_n, ln_a_w, ln_a_b, W_pr, b_pr, W_rf, b_rf, W_fc,
           b_fc):
    B, Cc, H, W = F_a.shape
    nband = H // (2 * PS)

    wfull = lambda s: pl.BlockSpec(s, lambda *_: tuple(0 for _ in s))
    row = lambda v: v.reshape(1, -1)

    def fa_idx(b, i):
        j = jnp.where(i < nband, i, jnp.maximum(i - nband - 1, 0))
        return (b, 0, j, 0)

    def out_idx(b, i):
        return (b, 0, jnp.maximum(i - nband - 1, 0), 0)

    out = pl.pallas_call(
        _fused_body,
        grid=(B, 2 * nband + 1),
        in_specs=[pl.BlockSpec((1, Cc, 2 * PS, W), fa_idx)]
        + [wfull(s) for s in [(Cc, 2 * Cc), (1, Cc), (Cc, Cc), (1, Cc),
                              (1, Cc), (1, Cc), (1, Cc), (1, Cc), (Cc, Cc),
                              (1, Cc), (1, Cc), (1, Cc), (1, Cc), (Cc, Cc),
                              (1, Cc), (Cc, Cc), (1, Cc), (Cc, Cc),
                              (Cc, 1)]],
        out_specs=pl.BlockSpec((1, Cc, 2 * PS, W), out_idx),
        out_shape=jax.ShapeDtypeStruct((B, Cc, H, W), jnp.float32),
        scratch_shapes=[
            pltpu.VMEM((NBLK, PROW, Cc), jnp.float32),
            pltpu.VMEM((NBLK, Cc, PROW), jnp.float32),
        ],
    )(F_a, W_e1, row(b_e1), W_e2, row(b_e2), row(ln_g_w), row(ln_g_b),
      W_px, jnp.broadcast_to(b_px.reshape(1, 1), (1, Cc)), W_np, row(b_np),
      lambda_n.reshape(1, Cc), row(ln_a_w), row(ln_a_b), W_pr, row(b_pr),
      W_rf, row(b_rf), W_fc.reshape(Cc, Cc), b_fc.reshape(Cc, 1))

    return out
